# Initial kernel scaffold; baseline (speedup 1.0000x reference)
import jax
import jax.numpy as jnp
from jax.experimental import pallas as pl


def kernel(h_playlist, h_track, src_idx, dst_idx):
    raise NotImplementedError("write your pallas kernel here")


# speedup vs baseline: 3.2030x; 3.2030x over previous
import functools

import jax
import jax.numpy as jnp
from jax import lax
from jax.experimental import pallas as pl
from jax.experimental.pallas import tpu as pltpu
from jax.experimental.pallas import tpu_sc as plsc

N_PLAYLIST = 10000
N_TRACK = 10000
N_EDGES = 320000
D_FEAT = 128

NUM_CORES = 2
NUM_SUBCORES = 16
NUM_WORKERS = NUM_CORES * NUM_SUBCORES
EDGES_PER_WORKER = N_EDGES // NUM_WORKERS
CHUNK = 400
NUM_CHUNKS = EDGES_PER_WORKER // CHUNK


def _sc_body(h_playlist, h_track, src_idx, dst_idx, out,
             idx_u, idx_v, u_rows, v_rows, out_v, sem_u, sem_v):
    wid = lax.axis_index("s") * NUM_CORES + lax.axis_index("c")
    worker_base = wid * EDGES_PER_WORKER

    def chunk_body(c, _):
        base = worker_base + c * CHUNK
        pltpu.sync_copy(src_idx.at[pl.ds(base, CHUNK)], idx_u)
        pltpu.sync_copy(dst_idx.at[pl.ds(base, CHUNK)], idx_v)
        cp_u = pltpu.async_copy(h_playlist.at[idx_u], u_rows, sem_u)
        cp_v = pltpu.async_copy(h_track.at[idx_v], v_rows, sem_v)
        cp_u.wait()
        cp_v.wait()

        lane = lax.iota(jnp.int32, 16)

        def group_body(g, _):
            e0 = g * 16
            res = jnp.zeros((16,), jnp.float32)
            for k in range(16):
                e = e0 + k
                acc = u_rows[e, pl.ds(0, 16)] * v_rows[e, pl.ds(0, 16)]
                for f in range(1, D_FEAT // 16):
                    acc = acc + (u_rows[e, pl.ds(f * 16, 16)]
                                 * v_rows[e, pl.ds(f * 16, 16)])
                res = jnp.where(lane == k, jnp.sum(acc), res)
            out_v[pl.ds(e0, 16)] = res
            return 0

        lax.fori_loop(0, CHUNK // 16, group_body, 0)
        pltpu.sync_copy(out_v, out.at[pl.ds(base, CHUNK)])
        return 0

    lax.fori_loop(0, NUM_CHUNKS, chunk_body, 0)


@jax.jit
def _scores(h_playlist, h_track, src_idx, dst_idx):
    mesh = plsc.VectorSubcoreMesh(core_axis_name="c", subcore_axis_name="s")
    return pl.kernel(
        _sc_body,
        out_type=jax.ShapeDtypeStruct((N_EDGES,), jnp.float32),
        mesh=mesh,
        compiler_params=pltpu.CompilerParams(needs_layout_passes=False),
        scratch_types=[
            pltpu.VMEM((CHUNK,), jnp.int32),
            pltpu.VMEM((CHUNK,), jnp.int32),
            pltpu.VMEM((CHUNK, D_FEAT), jnp.float32),
            pltpu.VMEM((CHUNK, D_FEAT), jnp.float32),
            pltpu.VMEM((CHUNK,), jnp.float32),
            pltpu.SemaphoreType.DMA,
            pltpu.SemaphoreType.DMA,
        ],
    )(h_playlist, h_track, src_idx, dst_idx)


def kernel(h_playlist, h_track, src_idx, dst_idx):
    return _scores(h_playlist, h_track, src_idx, dst_idx).reshape(N_EDGES, 1)


# speedup vs baseline: 4.1962x; 1.3101x over previous
import functools

import jax
import jax.numpy as jnp
from jax import lax
from jax.experimental import pallas as pl
from jax.experimental.pallas import tpu as pltpu
from jax.experimental.pallas import tpu_sc as plsc

N_PLAYLIST = 10000
N_TRACK = 10000
N_EDGES = 320000
D_FEAT = 128

NUM_CORES = 2
NUM_SUBCORES = 16
NUM_WORKERS = NUM_CORES * NUM_SUBCORES
EDGES_PER_WORKER = N_EDGES // NUM_WORKERS
CHUNK = 80
NUM_CHUNKS = EDGES_PER_WORKER // CHUNK
NUM_PAIRS = (NUM_CHUNKS - 1) // 2


def _sc_body(h_playlist, h_track, src_idx, dst_idx, out,
             idx_u, idx_v, u0, v0, u1, v1, out_v,
             su0, sv0, su1, sv1):
    wid = lax.axis_index("s") * NUM_CORES + lax.axis_index("c")
    worker_base = wid * EDGES_PER_WORKER

    pltpu.sync_copy(src_idx.at[pl.ds(worker_base, EDGES_PER_WORKER)], idx_u)
    pltpu.sync_copy(dst_idx.at[pl.ds(worker_base, EDGES_PER_WORKER)], idx_v)

    def start(c, u_rows, v_rows, su, sv):
        iu = idx_u.at[pl.ds(c * CHUNK, CHUNK)]
        iv = idx_v.at[pl.ds(c * CHUNK, CHUNK)]
        pltpu.async_copy(h_playlist.at[iu], u_rows, su)
        pltpu.async_copy(h_track.at[iv], v_rows, sv)

    def wait(c, u_rows, v_rows, su, sv):
        iu = idx_u.at[pl.ds(c * CHUNK, CHUNK)]
        iv = idx_v.at[pl.ds(c * CHUNK, CHUNK)]
        pltpu.make_async_copy(h_playlist.at[iu], u_rows, su).wait()
        pltpu.make_async_copy(h_track.at[iv], v_rows, sv).wait()

    lane = lax.iota(jnp.int32, 16)

    def compute(c, u_rows, v_rows):
        def group_body(g, _):
            e0 = g * 16
            res = jnp.zeros((16,), jnp.float32)
            for k in range(16):
                e = e0 + k
                acc = u_rows[e, pl.ds(0, 16)] * v_rows[e, pl.ds(0, 16)]
                for f in range(1, D_FEAT // 16):
                    acc = acc + (u_rows[e, pl.ds(f * 16, 16)]
                                 * v_rows[e, pl.ds(f * 16, 16)])
                res = jnp.where(lane == k, jnp.sum(acc), res)
            out_v[pl.ds(e0, 16)] = res
            return 0

        lax.fori_loop(0, CHUNK // 16, group_body, 0)
        pltpu.sync_copy(out_v, out.at[pl.ds(worker_base + c * CHUNK, CHUNK)])

    start(0, u0, v0, su0, sv0)

    def pair_body(g, _):
        c = 2 * g
        start(c + 1, u1, v1, su1, sv1)
        wait(c, u0, v0, su0, sv0)
        compute(c, u0, v0)
        start(c + 2, u0, v0, su0, sv0)
        wait(c + 1, u1, v1, su1, sv1)
        compute(c + 1, u1, v1)
        return 0

    lax.fori_loop(0, NUM_PAIRS, pair_body, 0)
    wait(NUM_CHUNKS - 1, u0, v0, su0, sv0)
    compute(NUM_CHUNKS - 1, u0, v0)


@jax.jit
def _scores(h_playlist, h_track, src_idx, dst_idx):
    mesh = plsc.VectorSubcoreMesh(core_axis_name="c", subcore_axis_name="s")
    return pl.kernel(
        _sc_body,
        out_type=jax.ShapeDtypeStruct((N_EDGES,), jnp.float32),
        mesh=mesh,
        compiler_params=pltpu.CompilerParams(needs_layout_passes=False),
        scratch_types=[
            pltpu.VMEM((EDGES_PER_WORKER,), jnp.int32),
            pltpu.VMEM((EDGES_PER_WORKER,), jnp.int32),
            pltpu.VMEM((CHUNK, D_FEAT), jnp.float32),
            pltpu.VMEM((CHUNK, D_FEAT), jnp.float32),
            pltpu.VMEM((CHUNK, D_FEAT), jnp.float32),
            pltpu.VMEM((CHUNK, D_FEAT), jnp.float32),
            pltpu.VMEM((CHUNK,), jnp.float32),
            pltpu.SemaphoreType.DMA,
            pltpu.SemaphoreType.DMA,
            pltpu.SemaphoreType.DMA,
            pltpu.SemaphoreType.DMA,
        ],
    )(h_playlist, h_track, src_idx, dst_idx)


def kernel(h_playlist, h_track, src_idx, dst_idx):
    return _scores(h_playlist, h_track, src_idx, dst_idx).reshape(N_EDGES, 1)
